# baseline (speedup 1.0000x reference)
import functools

import jax
import jax.numpy as jnp
from jax import lax
from jax.experimental import pallas as pl
from jax.experimental.pallas import tpu as pltpu
from jax.experimental.pallas import tpu_sc as plsc

NUM_SETS = 1000000
EMBED_DIM = 64
HIDDEN_DIM = 128
BATCH = 16384
TEAM = 6

NC, NS = 2, 16
NW = NC * NS
TEAMS_PER_W = BATCH // NW
TEAMS_PER_CHUNK = 16
CHUNKS = TEAMS_PER_W // TEAMS_PER_CHUNK
IDX_PER_CHUNK = TEAMS_PER_CHUNK * TEAM
LANES = 16


def _sc_pool(idx3, table):
    mesh = plsc.VectorSubcoreMesh(core_axis_name="c", subcore_axis_name="s")

    @functools.partial(
        pl.kernel,
        out_type=jax.ShapeDtypeStruct((BATCH, EMBED_DIM), jnp.float32),
        mesh=mesh,
        scratch_types=[
            pltpu.VMEM((CHUNKS, IDX_PER_CHUNK), jnp.int32),
            pltpu.VMEM((IDX_PER_CHUNK, EMBED_DIM), jnp.float32),
            pltpu.VMEM((TEAMS_PER_W, EMBED_DIM), jnp.float32),
            pltpu.SemaphoreType.DMA,
        ],
        compiler_params=pltpu.CompilerParams(use_tc_tiling_on_sc=False),
    )
    def k(idx_hbm, table_hbm, out_hbm, idx_v, rows_v, out_v, sem):
        wid = lax.axis_index("s") * NC + lax.axis_index("c")
        pltpu.sync_copy(idx_hbm.at[wid], idx_v)

        def chunk_body(j, carry):
            pltpu.async_copy(table_hbm.at[idx_v.at[j]], rows_v, sem).wait()

            def team_body(t, c2):
                row0 = t * TEAM
                orow = j * TEAMS_PER_CHUNK + t
                for cb in range(EMBED_DIM // LANES):
                    sl = pl.ds(cb * LANES, LANES)
                    acc = rows_v[row0, sl]
                    for r in range(1, TEAM):
                        acc = acc + rows_v[row0 + r, sl]
                    out_v[orow, sl] = acc * (1.0 / TEAM)
                return c2

            lax.fori_loop(0, TEAMS_PER_CHUNK, team_body, 0)
            return carry

        lax.fori_loop(0, CHUNKS, chunk_body, 0)
        pltpu.sync_copy(out_v, out_hbm.at[pl.ds(wid * TEAMS_PER_W, TEAMS_PER_W)])

    return k(idx3, table)


def _tc_mlp(x, w1t, b1, w2t, b2):
    bb = 1024

    def body(x_ref, w1_ref, b1_ref, w2_ref, b2_ref, o_ref):
        h = jnp.dot(x_ref[...], w1_ref[...], preferred_element_type=jnp.float32)
        h = jnp.maximum(h + b1_ref[...], 0.0)
        o_ref[...] = (
            jnp.dot(h, w2_ref[...], preferred_element_type=jnp.float32) + b2_ref[...]
        )

    return pl.pallas_call(
        body,
        grid=(BATCH // bb,),
        in_specs=[
            pl.BlockSpec((bb, EMBED_DIM), lambda i: (i, 0)),
            pl.BlockSpec((EMBED_DIM, HIDDEN_DIM), lambda i: (0, 0)),
            pl.BlockSpec((1, HIDDEN_DIM), lambda i: (0, 0)),
            pl.BlockSpec((HIDDEN_DIM, 1), lambda i: (0, 0)),
            pl.BlockSpec((1, 1), lambda i: (0, 0)),
        ],
        out_specs=pl.BlockSpec((bb, 1), lambda i: (i, 0)),
        out_shape=jax.ShapeDtypeStruct((BATCH, 1), jnp.float32),
    )(x, w1t, b1, w2t, b2)


def kernel(team_indices, embedding, fc1_w, fc1_b, fc2_w, fc2_b):
    idx3 = team_indices.astype(jnp.int32).reshape(NW, CHUNKS, IDX_PER_CHUNK)
    pooled = _sc_pool(idx3, embedding)
    out = _tc_mlp(
        pooled,
        fc1_w.T,
        fc1_b.reshape(1, HIDDEN_DIM),
        fc2_w.T,
        fc2_b.reshape(1, 1),
    )
    return out[:, 0]
